# baseline (speedup 1.0000x reference)
import functools
import math

import jax
import jax.numpy as jnp
from jax import lax
from jax.experimental import pallas as pl
from jax.experimental.pallas import tpu as pltpu
from jax.experimental.pallas import tpu_sc as plsc

N = 2048
D = 1024
H = 16
HD = 64
K = 8
RB = 256
SCALE = 1.0 / math.sqrt(HD)
LANES = 16

NC = 2
NS = 16
NW = NC * NS
CQ = 4
CR = CQ * K

CH = 2
NRC = N // CH


def _proj_body(h_ref, wk_ref, wv_ref, wo_ref, kt_ref, u_ref):
    hb = h_ref[...]
    dn = (((1,), (1,)), ((), ()))
    kt_ref[...] = lax.dot_general(wk_ref[...], hb, dn,
                                  preferred_element_type=jnp.float32)
    vb = lax.dot_general(hb, wv_ref[...], dn,
                         preferred_element_type=jnp.float32)
    u_ref[...] = lax.dot_general(vb, wo_ref[...], dn,
                                 preferred_element_type=jnp.float32)


def _proj(h2, Wk, Wv, Wo):
    grid = N // RB
    return pl.pallas_call(
        _proj_body,
        grid=(grid,),
        in_specs=[
            pl.BlockSpec((RB, D), lambda i: (i, 0)),
            pl.BlockSpec((D, D), lambda i: (0, 0)),
            pl.BlockSpec((D, D), lambda i: (0, 0)),
            pl.BlockSpec((D, D), lambda i: (0, 0)),
        ],
        out_specs=[
            pl.BlockSpec((D, RB), lambda i: (0, i)),
            pl.BlockSpec((RB, D), lambda i: (i, 0)),
        ],
        out_shape=[
            jax.ShapeDtypeStruct((D, N), jnp.float32),
            jax.ShapeDtypeStruct((N, D), jnp.float32),
        ],
        compiler_params=pltpu.CompilerParams(
            dimension_semantics=("arbitrary",)),
    )(h2, Wk, Wv, Wo)


def _score_topk_body(h_ref, wq_ref, kt_ref, idx_ref, p_ref, pb_ref):
    dn = (((1,), (1,)), ((), ()))
    q2 = lax.dot_general(h_ref[...], wq_ref[...], dn,
                         preferred_element_type=jnp.float32) * jnp.float32(
                             SCALE / H)
    s = None
    for hh in range(H):
        qh = q2[:, hh * HD:(hh + 1) * HD]
        kh = kt_ref[hh * HD:(hh + 1) * HD, :]
        ph = lax.dot_general(qh, kh, (((1,), (0,)), ((), ())),
                             preferred_element_type=jnp.float32)
        ph = jnp.clip(ph, -10.0 / H, 10.0 / H)
        s = ph if s is None else s + ph

    col = lax.broadcasted_iota(jnp.int32, (RB, N), 1)
    vals, idxs = [], []
    for _ in range(K):
        m = jnp.max(s, axis=1, keepdims=True)
        cand = jnp.where(s == m, col, N)
        a = jnp.min(cand, axis=1, keepdims=True)
        vals.append(m)
        idxs.append(a)
        s = jnp.where(col == a, jnp.float32(-3.0e38), s)

    v = jnp.concatenate(vals, axis=1)
    i = jnp.concatenate(idxs, axis=1)
    vc = jnp.clip(v, -5.0, 5.0)
    e = jnp.exp(vc - jnp.max(vc, axis=1, keepdims=True))
    p = e / jnp.sum(e, axis=1, keepdims=True)

    idx_ref[...] = i
    p_ref[...] = p
    pb_ref[...] = jnp.broadcast_to(
        p[:, :, None], (RB, K, LANES)).reshape(RB, K * LANES)


def _score_topk(hs, Wq, kt):
    grid = NRC // RB
    return pl.pallas_call(
        _score_topk_body,
        grid=(grid,),
        in_specs=[
            pl.BlockSpec((RB, D), lambda i: (i, 0)),
            pl.BlockSpec((D, D), lambda i: (0, 0)),
            pl.BlockSpec((D, N), lambda i: (0, 0)),
        ],
        out_specs=[
            pl.BlockSpec((RB, K), lambda i: (i, 0)),
            pl.BlockSpec((RB, K), lambda i: (i, 0)),
            pl.BlockSpec((RB, K * LANES), lambda i: (i, 0)),
        ],
        out_shape=[
            jax.ShapeDtypeStruct((NRC, K), jnp.int32),
            jax.ShapeDtypeStruct((NRC, K), jnp.float32),
            jax.ShapeDtypeStruct((NRC, K * LANES), jnp.float32),
        ],
        compiler_params=pltpu.CompilerParams(
            dimension_semantics=("arbitrary",)),
    )(hs, Wq, kt)


def _gather_body(u_hbm, idx_hbm, pb_hbm, z_hbm, idx_v, pb_v, rows_v, out_v,
                 sem0, sem1):
    qw = NRC // NW
    nch = qw // CQ
    wid = lax.axis_index("s") * NC + lax.axis_index("c")
    qbase = wid * qw

    pltpu.sync_copy(idx_hbm.at[pl.ds(qbase * K, qw * K)], idx_v)
    pltpu.sync_copy(pb_hbm.at[pl.ds(qbase * K * LANES, qw * K * LANES)], pb_v)

    def start(c, buf, sem):
        pltpu.async_copy(u_hbm.at[idx_v.at[pl.ds(c * CR, CR)]],
                         rows_v.at[buf], sem)

    def wait(c, buf, sem):
        pltpu.make_async_copy(u_hbm.at[idx_v.at[pl.ds(c * CR, CR)]],
                              rows_v.at[buf], sem).wait()

    def compute(c, buf):
        for ql in range(CQ):
            sp = [pb_v[pl.ds(((c * CQ + ql) * K + j) * LANES, LANES)]
                  for j in range(K)]

            @plsc.parallel_loop(0, D // LANES, unroll=8)
            def _e(e, _sp=sp, _ql=ql, _buf=buf):
                off = pl.ds(e * LANES, LANES)
                acc = _sp[0] * rows_v[_buf, _ql * K, off]
                for j in range(1, K):
                    acc = acc + _sp[j] * rows_v[_buf, _ql * K + j, off]
                out_v[_ql, off] = acc

        pltpu.sync_copy(out_v, z_hbm.at[pl.ds(qbase + c * CQ, CQ)])

    start(0, 0, sem0)

    def pair_body(c2, _):
        c = c2 * 2
        wait(c, 0, sem0)
        start(c + 1, 1, sem1)
        compute(c, 0)
        wait(c + 1, 1, sem1)

        @pl.when(c2 + 1 < nch // 2)
        def _():
            start(c + 2, 0, sem0)

        compute(c + 1, 1)
        return 0

    lax.fori_loop(0, nch // 2, pair_body, 0)


@functools.cache
def _gather_kernel():
    qw = NRC // NW
    return pl.kernel(
        _gather_body,
        out_type=jax.ShapeDtypeStruct((NRC, D), jnp.float32),
        mesh=plsc.VectorSubcoreMesh(core_axis_name="c", subcore_axis_name="s",
                                    num_cores=NC, num_subcores=NS),
        scratch_types=[
            pltpu.VMEM((qw * K,), jnp.int32),
            pltpu.VMEM((qw * K * LANES,), jnp.float32),
            pltpu.VMEM((2, CR, D), jnp.float32),
            pltpu.VMEM((CQ, D), jnp.float32),
            pltpu.SemaphoreType.DMA,
            pltpu.SemaphoreType.DMA,
        ],
    )


def kernel(h, Wq, Wk, Wv, Wo):
    h2 = h.reshape(N, D)
    kt, u = _proj(h2, Wk, Wv, Wo)
    idxs, ps, zs = [], [], []
    for c in range(CH):
        hs = h2[c * NRC:(c + 1) * NRC]
        idx_c, p_c, pb_c = _score_topk(hs, Wq, kt)
        z_c = _gather_kernel()(u, idx_c.reshape(-1), pb_c.reshape(-1))
        idxs.append(idx_c)
        ps.append(p_c)
        zs.append(z_c)
    idx = jnp.concatenate(idxs, axis=0)
    p = jnp.concatenate(ps, axis=0)
    z2 = jnp.concatenate(zs, axis=0)
    return z2[None], idx[None], p[None]


# speedup vs baseline: 1.1092x; 1.1092x over previous
import functools
import math

import jax
import jax.numpy as jnp
from jax import lax
from jax.experimental import pallas as pl
from jax.experimental.pallas import tpu as pltpu
from jax.experimental.pallas import tpu_sc as plsc

N = 2048
D = 1024
H = 16
HD = 64
K = 8
RB = 256
SCALE = 1.0 / math.sqrt(HD)
LANES = 16

NC = 2
NS = 16
NW = NC * NS
QW = N // NW
CQ = 4
CR = CQ * K
NCH = QW // CQ


def _tc_body(hf_ref, h_ref, wq_ref, wk_ref, wv_ref, wo_ref,
             idx_ref, p_ref, pb_ref, u_ref, kt_ref):
    i = pl.program_id(0)
    dn = (((1,), (1,)), ((), ()))

    @pl.when(i == 0)
    def _():
        kt_ref[...] = lax.dot_general(wk_ref[...], hf_ref[...], dn,
                                      preferred_element_type=jnp.float32)

    hb = h_ref[...]
    vb = lax.dot_general(hb, wv_ref[...], dn,
                         preferred_element_type=jnp.float32)
    u_ref[...] = lax.dot_general(vb, wo_ref[...], dn,
                                 preferred_element_type=jnp.float32)

    q2 = lax.dot_general(hb, wq_ref[...], dn,
                         preferred_element_type=jnp.float32) * jnp.float32(
                             SCALE / H)
    s = None
    for hh in range(H):
        qh = q2[:, hh * HD:(hh + 1) * HD]
        kh = kt_ref[hh * HD:(hh + 1) * HD, :]
        ph = lax.dot_general(qh, kh, (((1,), (0,)), ((), ())),
                             preferred_element_type=jnp.float32)
        ph = jnp.clip(ph, -10.0 / H, 10.0 / H)
        s = ph if s is None else s + ph

    col = lax.broadcasted_iota(jnp.int32, (RB, N), 1)
    vals, idxs = [], []
    for _ in range(K):
        m = jnp.max(s, axis=1, keepdims=True)
        cand = jnp.where(s == m, col, N)
        a = jnp.min(cand, axis=1, keepdims=True)
        vals.append(m)
        idxs.append(a)
        s = jnp.where(col == a, jnp.float32(-3.0e38), s)

    v = jnp.concatenate(vals, axis=1)
    i32 = jnp.concatenate(idxs, axis=1)
    vc = jnp.clip(v, -5.0, 5.0)
    e = jnp.exp(vc - jnp.max(vc, axis=1, keepdims=True))
    p = e / jnp.sum(e, axis=1, keepdims=True)

    idx_ref[...] = i32
    p_ref[...] = p
    pb_ref[...] = jnp.broadcast_to(
        p[:, :, None], (RB, K, LANES)).reshape(RB, K * LANES)


def _tc_stage(h2, Wq, Wk, Wv, Wo):
    grid = N // RB
    return pl.pallas_call(
        _tc_body,
        grid=(grid,),
        in_specs=[
            pl.BlockSpec((N, D), lambda i: (0, 0)),
            pl.BlockSpec((RB, D), lambda i: (i, 0)),
            pl.BlockSpec((D, D), lambda i: (0, 0)),
            pl.BlockSpec((D, D), lambda i: (0, 0)),
            pl.BlockSpec((D, D), lambda i: (0, 0)),
            pl.BlockSpec((D, D), lambda i: (0, 0)),
        ],
        out_specs=[
            pl.BlockSpec((RB, K), lambda i: (i, 0)),
            pl.BlockSpec((RB, K), lambda i: (i, 0)),
            pl.BlockSpec((RB, K * LANES), lambda i: (i, 0)),
            pl.BlockSpec((RB, D), lambda i: (i, 0)),
        ],
        out_shape=[
            jax.ShapeDtypeStruct((N, K), jnp.int32),
            jax.ShapeDtypeStruct((N, K), jnp.float32),
            jax.ShapeDtypeStruct((N, K * LANES), jnp.float32),
            jax.ShapeDtypeStruct((N, D), jnp.float32),
        ],
        scratch_shapes=[pltpu.VMEM((D, N), jnp.float32)],
        compiler_params=pltpu.CompilerParams(
            dimension_semantics=("arbitrary",)),
    )(h2, h2, Wq, Wk, Wv, Wo)


def _gather_body(u_hbm, idx_hbm, pb_hbm, z_hbm, idx_v, pb_v, rows_v, out_v,
                 sem0, sem1):
    wid = lax.axis_index("s") * NC + lax.axis_index("c")
    qbase = wid * QW

    pltpu.sync_copy(idx_hbm.at[pl.ds(qbase * K, QW * K)], idx_v)
    pltpu.sync_copy(pb_hbm.at[pl.ds(qbase * K * LANES, QW * K * LANES)], pb_v)

    def start(c, buf, sem):
        pltpu.async_copy(u_hbm.at[idx_v.at[pl.ds(c * CR, CR)]],
                         rows_v.at[buf], sem)

    def wait(c, buf, sem):
        pltpu.make_async_copy(u_hbm.at[idx_v.at[pl.ds(c * CR, CR)]],
                              rows_v.at[buf], sem).wait()

    def compute(c, buf):
        for ql in range(CQ):
            sp = [pb_v[pl.ds(((c * CQ + ql) * K + j) * LANES, LANES)]
                  for j in range(K)]

            @plsc.parallel_loop(0, D // LANES, unroll=8)
            def _e(e, _sp=sp, _ql=ql, _buf=buf):
                off = pl.ds(e * LANES, LANES)
                acc = _sp[0] * rows_v[_buf, _ql * K, off]
                for j in range(1, K):
                    acc = acc + _sp[j] * rows_v[_buf, _ql * K + j, off]
                out_v[_ql, off] = acc

        pltpu.sync_copy(out_v, z_hbm.at[pl.ds(qbase + c * CQ, CQ)])

    start(0, 0, sem0)

    def pair_body(c2, _):
        c = c2 * 2
        wait(c, 0, sem0)
        start(c + 1, 1, sem1)
        compute(c, 0)
        wait(c + 1, 1, sem1)

        @pl.when(c2 + 1 < NCH // 2)
        def _():
            start(c + 2, 0, sem0)

        compute(c + 1, 1)
        return 0

    lax.fori_loop(0, NCH // 2, pair_body, 0)


@functools.cache
def _gather_kernel():
    return pl.kernel(
        _gather_body,
        out_type=jax.ShapeDtypeStruct((N, D), jnp.float32),
        mesh=plsc.VectorSubcoreMesh(core_axis_name="c", subcore_axis_name="s",
                                    num_cores=NC, num_subcores=NS),
        scratch_types=[
            pltpu.VMEM((QW * K,), jnp.int32),
            pltpu.VMEM((QW * K * LANES,), jnp.float32),
            pltpu.VMEM((2, CR, D), jnp.float32),
            pltpu.VMEM((CQ, D), jnp.float32),
            pltpu.SemaphoreType.DMA,
            pltpu.SemaphoreType.DMA,
        ],
    )


def kernel(h, Wq, Wk, Wv, Wo):
    h2 = h.reshape(N, D)
    idx, p, pb, u = _tc_stage(h2, Wq, Wk, Wv, Wo)
    z2 = _gather_kernel()(u, idx.reshape(-1), pb.reshape(-1))
    return z2[None], idx[None], p[None]


# speedup vs baseline: 1.1164x; 1.0065x over previous
import functools
import math

import jax
import jax.numpy as jnp
from jax import lax
from jax.experimental import pallas as pl
from jax.experimental.pallas import tpu as pltpu
from jax.experimental.pallas import tpu_sc as plsc

N = 2048
D = 1024
H = 16
HD = 64
K = 8
RB = 256
SCALE = 1.0 / math.sqrt(HD)
LANES = 16

NC = 2
NS = 16
NW = NC * NS
QW = N // NW
CQ = 4
CR = CQ * K
NCH = QW // CQ


def _tc_body(hf_ref, h_ref, wq_ref, wk_ref, wv_ref, wo_ref,
             idx_ref, p_ref, pb_ref, u_ref, kt_ref):
    i = pl.program_id(0)
    dn = (((1,), (1,)), ((), ()))

    @pl.when(i == 0)
    def _():
        kt_ref[...] = lax.dot_general(wk_ref[...], hf_ref[...], dn,
                                      preferred_element_type=jnp.float32)

    hb = h_ref[...]
    vb = lax.dot_general(hb, wv_ref[...], dn,
                         preferred_element_type=jnp.float32)
    ub = lax.dot_general(vb, wo_ref[...], dn,
                         preferred_element_type=jnp.float32)
    lo = lax.bitcast_convert_type(ub[:, :D // 2].astype(jnp.bfloat16),
                                  jnp.uint16).astype(jnp.int32)
    hi = lax.bitcast_convert_type(ub[:, D // 2:].astype(jnp.bfloat16),
                                  jnp.uint16).astype(jnp.int32)
    u_ref[...] = lo | lax.shift_left(hi, 16)

    q2 = lax.dot_general(hb, wq_ref[...], dn,
                         preferred_element_type=jnp.float32) * jnp.float32(
                             SCALE / H)
    s = None
    for hh in range(H):
        qh = q2[:, hh * HD:(hh + 1) * HD]
        kh = kt_ref[hh * HD:(hh + 1) * HD, :]
        ph = lax.dot_general(qh, kh, (((1,), (0,)), ((), ())),
                             preferred_element_type=jnp.float32)
        ph = jnp.clip(ph, -10.0 / H, 10.0 / H)
        s = ph if s is None else s + ph

    col = lax.broadcasted_iota(jnp.int32, (RB, N), 1)
    vals, idxs = [], []
    for _ in range(K):
        m = jnp.max(s, axis=1, keepdims=True)
        cand = jnp.where(s == m, col, N)
        a = jnp.min(cand, axis=1, keepdims=True)
        vals.append(m)
        idxs.append(a)
        s = jnp.where(col == a, jnp.float32(-3.0e38), s)

    v = jnp.concatenate(vals, axis=1)
    i32 = jnp.concatenate(idxs, axis=1)
    vc = jnp.clip(v, -5.0, 5.0)
    e = jnp.exp(vc - jnp.max(vc, axis=1, keepdims=True))
    p = e / jnp.sum(e, axis=1, keepdims=True)

    idx_ref[...] = i32
    p_ref[...] = p
    pb_ref[...] = jnp.broadcast_to(
        p[:, :, None], (RB, K, LANES)).reshape(RB, K * LANES)


def _tc_stage(h2, Wq, Wk, Wv, Wo):
    grid = N // RB
    return pl.pallas_call(
        _tc_body,
        grid=(grid,),
        in_specs=[
            pl.BlockSpec((N, D), lambda i: (0, 0)),
            pl.BlockSpec((RB, D), lambda i: (i, 0)),
            pl.BlockSpec((D, D), lambda i: (0, 0)),
            pl.BlockSpec((D, D), lambda i: (0, 0)),
            pl.BlockSpec((D, D), lambda i: (0, 0)),
            pl.BlockSpec((D, D), lambda i: (0, 0)),
        ],
        out_specs=[
            pl.BlockSpec((RB, K), lambda i: (i, 0)),
            pl.BlockSpec((RB, K), lambda i: (i, 0)),
            pl.BlockSpec((RB, K * LANES), lambda i: (i, 0)),
            pl.BlockSpec((RB, D // 2), lambda i: (i, 0)),
        ],
        out_shape=[
            jax.ShapeDtypeStruct((N, K), jnp.int32),
            jax.ShapeDtypeStruct((N, K), jnp.float32),
            jax.ShapeDtypeStruct((N, K * LANES), jnp.float32),
            jax.ShapeDtypeStruct((N, D // 2), jnp.int32),
        ],
        scratch_shapes=[pltpu.VMEM((D, N), jnp.float32)],
        compiler_params=pltpu.CompilerParams(
            dimension_semantics=("arbitrary",)),
    )(h2, h2, Wq, Wk, Wv, Wo)


def _gather_body(u_hbm, idx_hbm, pb_hbm, z_hbm, idx_v, pb_v, rows_v, out_v,
                 sem0, sem1):
    wid = lax.axis_index("s") * NC + lax.axis_index("c")
    qbase = wid * QW

    pltpu.sync_copy(idx_hbm.at[pl.ds(qbase * K, QW * K)], idx_v)
    pltpu.sync_copy(pb_hbm.at[pl.ds(qbase * K * LANES, QW * K * LANES)], pb_v)

    def start(c, buf, sem):
        pltpu.async_copy(u_hbm.at[idx_v.at[pl.ds(c * CR, CR)]],
                         rows_v.at[buf], sem)

    def wait(c, buf, sem):
        pltpu.make_async_copy(u_hbm.at[idx_v.at[pl.ds(c * CR, CR)]],
                              rows_v.at[buf], sem).wait()

    def compute(c, buf):
        for ql in range(CQ):
            sp = [pb_v[pl.ds(((c * CQ + ql) * K + j) * LANES, LANES)]
                  for j in range(K)]

            @plsc.parallel_loop(0, D // (2 * LANES), unroll=8)
            def _e(e, _sp=sp, _ql=ql, _buf=buf):
                acc_a = None
                acc_b = None
                for j in range(K):
                    wi = rows_v[_buf, _ql * K + j, pl.ds(e * LANES, LANES)]
                    a = lax.bitcast_convert_type(lax.shift_left(wi, 16),
                                                 jnp.float32)
                    b = lax.bitcast_convert_type(wi & jnp.int32(-65536),
                                                 jnp.float32)
                    ta = _sp[j] * a
                    tb = _sp[j] * b
                    acc_a = ta if acc_a is None else acc_a + ta
                    acc_b = tb if acc_b is None else acc_b + tb
                out_v[_ql, pl.ds(e * LANES, LANES)] = acc_a
                out_v[_ql, pl.ds(D // 2 + e * LANES, LANES)] = acc_b

        pltpu.sync_copy(out_v, z_hbm.at[pl.ds(qbase + c * CQ, CQ)])

    start(0, 0, sem0)

    def pair_body(c2, _):
        c = c2 * 2
        wait(c, 0, sem0)
        start(c + 1, 1, sem1)
        compute(c, 0)
        wait(c + 1, 1, sem1)

        @pl.when(c2 + 1 < NCH // 2)
        def _():
            start(c + 2, 0, sem0)

        compute(c + 1, 1)
        return 0

    lax.fori_loop(0, NCH // 2, pair_body, 0)


@functools.cache
def _gather_kernel():
    return pl.kernel(
        _gather_body,
        out_type=jax.ShapeDtypeStruct((N, D), jnp.float32),
        mesh=plsc.VectorSubcoreMesh(core_axis_name="c", subcore_axis_name="s",
                                    num_cores=NC, num_subcores=NS),
        scratch_types=[
            pltpu.VMEM((QW * K,), jnp.int32),
            pltpu.VMEM((QW * K * LANES,), jnp.float32),
            pltpu.VMEM((2, CR, D // 2), jnp.int32),
            pltpu.VMEM((CQ, D), jnp.float32),
            pltpu.SemaphoreType.DMA,
            pltpu.SemaphoreType.DMA,
        ],
    )


def kernel(h, Wq, Wk, Wv, Wo):
    h2 = h.reshape(N, D)
    idx, p, pb, u = _tc_stage(h2, Wq, Wk, Wv, Wo)
    z2 = _gather_kernel()(u, idx.reshape(-1), pb.reshape(-1))
    return z2[None], idx[None], p[None]
